# baseline (speedup 1.0000x reference)
import functools

import jax
import jax.numpy as jnp
from jax import lax
from jax.experimental import pallas as pl
from jax.experimental.pallas import tpu as pltpu
from jax.experimental.pallas import tpu_sc as plsc

B = 16384
D = 64
NUM_AGE = 8
NUM_GENDER = 2
NUM_OCC = 21
NUM_GENRES = 19

_info = plsc.get_sparse_core_info()
NC, NS, L = _info.num_cores, _info.num_subcores, _info.num_lanes
NW = NC * NS
BPW = B // NW
NG = BPW // L


def _sc_gather_kernel():
    mesh = plsc.VectorSubcoreMesh(core_axis_name="c", subcore_axis_name="s")

    @functools.partial(
        pl.kernel,
        mesh=mesh,
        compiler_params=pltpu.CompilerParams(
            needs_layout_passes=False, use_tc_tiling_on_sc=False),
        out_type=[
            jax.ShapeDtypeStruct((B, D), jnp.float32),
            jax.ShapeDtypeStruct((B,), jnp.float32),
        ],
        scratch_types=[
            pltpu.VMEM((BPW,), jnp.int32),
            pltpu.VMEM((BPW,), jnp.int32),
            pltpu.VMEM((BPW,), jnp.int32),
            pltpu.VMEM((BPW,), jnp.int32),
            pltpu.VMEM((BPW,), jnp.int32),
            pltpu.VMEM((BPW, D), jnp.float32),
            pltpu.VMEM((BPW, D), jnp.float32),
            pltpu.VMEM((BPW,), jnp.float32),
            pltpu.VMEM((BPW,), jnp.float32),
            pltpu.VMEM((NUM_AGE, D), jnp.float32),
            pltpu.VMEM((NUM_GENDER, D), jnp.float32),
            pltpu.VMEM((NUM_OCC, D), jnp.float32),
            pltpu.VMEM((BPW,), jnp.float32),
            pltpu.SemaphoreType.DMA,
            pltpu.SemaphoreType.DMA,
            pltpu.SemaphoreType.DMA,
            pltpu.SemaphoreType.DMA,
        ],
    )
    def sc_kernel(uidx_hbm, midx_hbm, aidx_hbm, gidx_hbm, oidx_hbm,
                  uemb_hbm, memb_hbm, aemb_hbm, gemb_hbm, oemb_hbm,
                  ubias_hbm, mbias_hbm,
                  u_out_hbm, s_out_hbm,
                  uidx_v, midx_v, aidx_v, gidx_v, oidx_v,
                  urows_v, mrows_v, ub_v, mb_v,
                  atab_v, gtab_v, otab_v, s_v,
                  sem0, sem1, sem2, sem3):
        wid = lax.axis_index("s") * NC + lax.axis_index("c")
        base = wid * BPW

        pltpu.sync_copy(uidx_hbm.at[pl.ds(base, BPW)], uidx_v)
        pltpu.sync_copy(midx_hbm.at[pl.ds(base, BPW)], midx_v)
        cp_u = pltpu.async_copy(uemb_hbm.at[uidx_v], urows_v, sem0)
        cp_m = pltpu.async_copy(memb_hbm.at[midx_v], mrows_v, sem1)
        cp_ub = pltpu.async_copy(ubias_hbm.at[uidx_v], ub_v, sem2)
        cp_mb = pltpu.async_copy(mbias_hbm.at[midx_v], mb_v, sem3)

        pltpu.sync_copy(aidx_hbm.at[pl.ds(base, BPW)], aidx_v)
        pltpu.sync_copy(gidx_hbm.at[pl.ds(base, BPW)], gidx_v)
        pltpu.sync_copy(oidx_hbm.at[pl.ds(base, BPW)], oidx_v)
        pltpu.sync_copy(aemb_hbm, atab_v)
        pltpu.sync_copy(gemb_hbm, gtab_v)
        pltpu.sync_copy(oemb_hbm, otab_v)

        cp_u.wait()
        cp_m.wait()
        cp_ub.wait()
        cp_mb.wait()

        def group_body(g, carry):
            rid = g * L + lax.iota(jnp.int32, L)
            z16 = jnp.zeros((L,), jnp.int32)
            ai = aidx_v[pl.ds(g * L, L)]
            gi = gidx_v[pl.ds(g * L, L)]
            oi = oidx_v[pl.ds(g * L, L)]
            acc = ub_v[pl.ds(g * L, L)] + mb_v[pl.ds(g * L, L)]
            for d in range(D):
                cd = jnp.full((L,), d, jnp.int32)
                uc = plsc.load_gather(urows_v, [rid, cd])
                uc = (uc
                      + plsc.load_gather(atab_v, [ai, cd])
                      + plsc.load_gather(gtab_v, [gi, cd])
                      + plsc.load_gather(otab_v, [oi, cd]))
                mc = plsc.load_gather(mrows_v, [rid, cd])
                plsc.store_scatter(urows_v, [rid, cd], uc)
                acc = acc + uc * mc
            s_v[pl.ds(g * L, L)] = acc
            return carry

        lax.fori_loop(0, NG, group_body, 0)

        pltpu.sync_copy(urows_v, u_out_hbm.at[pl.ds(base, BPW)])
        pltpu.sync_copy(s_v, s_out_hbm.at[pl.ds(base, BPW)])

    return sc_kernel


_BK = 2048


def _tc_body(gb_ref, u_ref, gv_ref, get_ref, s_ref, o_ref):
    t = jnp.dot(u_ref[...], get_ref[...], preferred_element_type=jnp.float32)
    o_ref[...] = s_ref[...] + gb_ref[0] + jnp.sum(t * gv_ref[...], axis=1)


def kernel(user_idx, movie_idx, occ_idx, age_idx, gender_idx, genre_vec,
           user_emb, movie_emb, age_emb, gender_emb, occ_emb, genre_emb,
           user_bias, movie_bias, global_bias):
    user_idx = user_idx.astype(jnp.int32)
    movie_idx = movie_idx.astype(jnp.int32)
    occ_idx = occ_idx.astype(jnp.int32)
    age_idx = age_idx.astype(jnp.int32)
    gender_idx = gender_idx.astype(jnp.int32)

    u_lat, s_part = _sc_gather_kernel()(
        user_idx, movie_idx, age_idx, gender_idx, occ_idx,
        user_emb, movie_emb, age_emb, gender_emb, occ_emb,
        user_bias.reshape(-1), movie_bias.reshape(-1))

    ge_t = genre_emb.T

    out = pl.pallas_call(
        _tc_body,
        grid=(B // _BK,),
        in_specs=[
            pl.BlockSpec(memory_space=pltpu.SMEM),
            pl.BlockSpec((_BK, D), lambda i: (i, 0)),
            pl.BlockSpec((_BK, NUM_GENRES), lambda i: (i, 0)),
            pl.BlockSpec((D, NUM_GENRES), lambda i: (0, 0)),
            pl.BlockSpec((_BK,), lambda i: (i,)),
        ],
        out_specs=pl.BlockSpec((_BK,), lambda i: (i,)),
        out_shape=jax.ShapeDtypeStruct((B,), jnp.float32),
    )(global_bias, u_lat, genre_vec, ge_t, s_part)
    return out
